# baseline (speedup 1.0000x reference)
import functools

import jax
import jax.numpy as jnp
import numpy as np
from jax import lax
from jax.experimental import pallas as pl
from jax.experimental.pallas import tpu as pltpu
from jax.experimental.pallas import tpu_sc as plsc

N = 10000
E = 320000
H = 128
NC = 2
NS = 16
NW = NC * NS
EPW = E // NW
CH = 64
NPAD = 10112
RPT = NPAD // NS
ZR = 16
WB = 5

_sc_mesh = plsc.VectorSubcoreMesh(core_axis_name="c", subcore_axis_name="s")


@functools.partial(
    pl.kernel,
    out_type=jax.ShapeDtypeStruct((NC, NPAD, H), jnp.float32),
    mesh=_sc_mesh,
    scratch_types=[
        pltpu.VMEM((3, CH), jnp.int32),
        pltpu.VMEM((4, CH), jnp.int32),
        pltpu.VMEM((8, 2048), jnp.float32),
        pltpu.VMEM((4, CH, H), jnp.float32),
        pltpu.VMEM_SHARED((NPAD, H), jnp.float32),
        pltpu.SemaphoreType.DMA,
        pltpu.SemaphoreType.DMA,
        pltpu.SemaphoreType.DMA,
        pltpu.SemaphoreType.DMA,
        pltpu.SemaphoreType.DMA,
    ],
)
def _sc_aggregate(x_hbm, ei_hbm, ew_hbm, out_hbm,
                  sidx_v, didx_v, w_v, rows_v, acc_sh, gsem, ssem, dsem, wsem,
                  csem):
    c = lax.axis_index("c")
    s = lax.axis_index("s")
    wid = s * NC + c
    g = jnp.where(wid < 24, 10 * wid, 240 + 9 * (wid - 24))
    nk = jnp.where(wid < 24, 160, 144)
    ebase = g * 1024
    rowbase = g * 8

    def sidx_load(k):
        return pltpu.async_copy(
            ei_hbm.at[0, pl.ds(ebase + k * CH, CH)],
            sidx_v.at[lax.rem(k, 3)], ssem)

    def didx_load(k):
        return pltpu.async_copy(
            ei_hbm.at[1, pl.ds(ebase + k * CH, CH)],
            didx_v.at[lax.rem(k, 4)], dsem)

    def gather(k):
        return pltpu.async_copy(
            x_hbm.at[sidx_v.at[lax.rem(k, 3)]],
            rows_v.at[lax.rem(k, 4)], gsem)

    def mul_rows(b, kk):
        def mul(e, _):
            wsp = w_v[kk // 2, pl.ds(lax.rem(kk, 2) * 1024 + e * 16, 16)]
            for j in range(8):
                rows_v[b, e, pl.ds(j * 16, 16)] = (
                    rows_v[b, e, pl.ds(j * 16, 16)] * wsp)
            return 0
        lax.fori_loop(0, CH, mul, 0, unroll=4)

    sidx_load(0).wait()
    gather(0)
    sidx_load(1)
    didx_load(0)

    def zb(k, _):
        rows_v[3, k // 8, pl.ds((k % 8) * 16, 16)] = jnp.zeros(
            (16,), jnp.float32)
        return 0
    lax.fori_loop(0, CH * 8, zb, 0)
    for t in range(9):
        pltpu.sync_copy(
            rows_v.at[3], acc_sh.at[pl.ds(s * RPT + t * 64, 64)])
    pltpu.sync_copy(
        rows_v.at[3, pl.ds(0, 56)], acc_sh.at[pl.ds(s * RPT + 576, 56)])
    plsc.subcore_barrier()

    def chunk(k, _):
        b = lax.rem(k, 4)
        @pl.when(lax.rem(k, 16) == 0)
        def _():
            pltpu.async_copy(
                ew_hbm.at[pl.ds(rowbase + (k // 16) * 8, 8)], w_v,
                wsem).wait()

        pltpu.make_async_copy(
            x_hbm.at[sidx_v.at[lax.rem(k, 3)]], rows_v.at[b], gsem).wait()

        @pl.when(k >= 3)
        def _():
            pltpu.make_async_copy(
                rows_v.at[lax.rem(k + 1, 4)],
                acc_sh.at[didx_v.at[lax.rem(k + 1, 4)]], csem).wait()
        @pl.when(k + 1 < nk)
        def _():
            pltpu.make_async_copy(
                ei_hbm.at[0, pl.ds(ebase + (k + 1) * CH, CH)],
                sidx_v.at[lax.rem(k + 1, 3)], ssem).wait()
            gather(k + 1)
            @pl.when(k + 2 < nk)
            def _():
                sidx_load(k + 2)

        mul_rows(b, lax.rem(k, 16))

        pltpu.make_async_copy(
            ei_hbm.at[1, pl.ds(ebase + k * CH, CH)],
            didx_v.at[b], dsem).wait()
        @pl.when(k + 1 < nk)
        def _():
            didx_load(k + 1)
        pltpu.async_copy(rows_v.at[b], acc_sh.at[didx_v.at[b]], csem,
                         add=True)
        return 0
    lax.fori_loop(0, nk, chunk, 0)
    for d in range(3):
        @pl.when(nk - 3 + d >= 0)
        def _(d=d):
            pltpu.make_async_copy(
                rows_v.at[lax.rem(nk - 3 + d, 4)],
                acc_sh.at[didx_v.at[lax.rem(nk - 3 + d, 4)]], csem).wait()

    @pl.when(wid < 8)
    def _():
        eoff = 319488 + wid * CH
        pltpu.async_copy(ew_hbm.at[pl.ds(2496, 8)], w_v, wsem).wait()
        pltpu.async_copy(
            ei_hbm.at[0, pl.ds(eoff, CH)], sidx_v.at[0], ssem).wait()
        pltpu.async_copy(
            ei_hbm.at[1, pl.ds(eoff, CH)], didx_v.at[0], dsem).wait()
        pltpu.async_copy(
            x_hbm.at[sidx_v.at[0]], rows_v.at[0], gsem).wait()
        mul_rows(0, wid)
        pltpu.sync_copy(rows_v.at[0], acc_sh.at[didx_v.at[0]], add=True)
    plsc.subcore_barrier()

    for t in range(9):
        r0 = s * RPT + t * 64
        pltpu.sync_copy(acc_sh.at[pl.ds(r0, 64)], rows_v.at[0])
        pltpu.sync_copy(rows_v.at[0], out_hbm.at[c, pl.ds(r0, 64)])
    r0 = s * RPT + 576
    pltpu.sync_copy(acc_sh.at[pl.ds(r0, 56)], rows_v.at[0, pl.ds(0, 56)])
    pltpu.sync_copy(
        rows_v.at[0, pl.ds(0, 56)], out_hbm.at[c, pl.ds(r0, 56)])


def _dotT(a, b):
    return lax.dot_general(a, b, (((1,), (1,)), ((), ())),
                           preferred_element_type=jnp.float32)


BLK = 1264
GRID = NPAD // BLK


def _first_body(x_ref, w_ref, b_ref, o_ref):
    o_ref[...] = jnp.maximum(_dotT(x_ref[...], w_ref[...]) + b_ref[...], 0.0)


EROWS = 2560


def _rep16_body(w_ref, r_ref, o_ref):
    o_ref[...] = lax.dot_general(
        w_ref[...], r_ref[...], (((1,), (0,)), ((), ())),
        preferred_element_type=jnp.float32,
        precision=lax.Precision.HIGHEST)


def _gru_update(h, a0, a1, wl, bl, wih, bih, whh, bhh, fw):
    agg = a0 + a1
    xl = _dotT(agg, wl) + bl
    gi = _dotT(xl, wih) + bih
    gh = _dotT(h, whh) + bhh
    r = jax.nn.sigmoid(gi[:, :H] + gh[:, :H])
    z = jax.nn.sigmoid(gi[:, H:2 * H] + gh[:, H:2 * H])
    n = jnp.tanh(gi[:, 2 * H:] + r * gh[:, 2 * H:])
    return (1.0 - z) * n + z * h + fw * h


def _layer_body(h_ref, a0_ref, a1_ref, wl_ref, bl_ref, wih_ref, bih_ref,
                whh_ref, bhh_ref, fw_ref, o_ref):
    o_ref[...] = _gru_update(h_ref[...], a0_ref[...], a1_ref[...],
                             wl_ref[...], bl_ref[...], wih_ref[...],
                             bih_ref[...], whh_ref[...], bhh_ref[...],
                             fw_ref[0, 0])


def _final_body(h_ref, a0_ref, a1_ref, wl_ref, bl_ref, wih_ref, bih_ref,
                whh_ref, bhh_ref, fw_ref, wout_ref, bout_ref, o_ref):
    hn = _gru_update(h_ref[...], a0_ref[...], a1_ref[...], wl_ref[...],
                     bl_ref[...], wih_ref[...], bih_ref[...], whh_ref[...],
                     bhh_ref[...], fw_ref[0, 0])
    logits = _dotT(hn, wout_ref[...]) + bout_ref[...]
    m = jnp.max(logits, axis=-1, keepdims=True)
    lse = m + jnp.log(jnp.sum(jnp.exp(logits - m), axis=-1, keepdims=True))
    o_ref[...] = logits - lse


def _row_spec(shape):
    nd = len(shape)
    return pl.BlockSpec((BLK,) + tuple(shape[1:]),
                        lambda i, nd=nd: (i,) + (0,) * (nd - 1))


def _full_spec(shape):
    nd = len(shape)
    return pl.BlockSpec(tuple(shape), lambda i, nd=nd: (0,) * nd)


def kernel(x, edge_index, edge_weight, W_first, b_first, W1_0, b1_0, W1_1, b1_1,
           W_ih, W_hh, b_ih, b_hh, fuse_weight, W_out, b_out):
    ew_pad = jnp.concatenate(
        [edge_weight, jnp.zeros((EROWS * 128 - E,), jnp.float32)])
    rep = (jnp.arange(2048, dtype=jnp.int32)[None, :] // 16
           == jnp.arange(128, dtype=jnp.int32)[:, None]).astype(jnp.float32)
    ew16 = pl.pallas_call(
        _rep16_body,
        grid=(EROWS // 160,),
        in_specs=[pl.BlockSpec((160, 128), lambda i: (i, 0)),
                  pl.BlockSpec((128, 2048), lambda i: (0, 0))],
        out_specs=pl.BlockSpec((160, 128 * 16), lambda i: (i, 0)),
        out_shape=jax.ShapeDtypeStruct((EROWS, 128 * 16), jnp.float32),
    )(ew_pad.reshape(EROWS, 128), rep)
    b_first2 = b_first.reshape(1, H)
    b_ih2 = b_ih.reshape(1, 3 * H)
    b_hh2 = b_hh.reshape(1, 3 * H)
    b_out2 = b_out.reshape(1, 2)

    x0 = pl.pallas_call(
        _first_body,
        grid=(GRID,),
        in_specs=[_row_spec((N, H)), _full_spec((H, H)), _full_spec((1, H))],
        out_specs=_row_spec((NPAD, H)),
        out_shape=jax.ShapeDtypeStruct((NPAD, H), jnp.float32),
    )(x, W_first, b_first2)

    layer_in_specs = [
        _row_spec((NPAD, H)),
        _row_spec((NPAD, H)),
        _row_spec((NPAD, H)),
        _full_spec((H, H)),
        _full_spec((1, H)),
        _full_spec((3 * H, H)),
        _full_spec((1, 3 * H)),
        _full_spec((3 * H, H)),
        _full_spec((1, 3 * H)),
        _full_spec((1, 1)),
    ]

    h = x0
    aggp = _sc_aggregate(h, edge_index, ew16)
    h = pl.pallas_call(
        _layer_body,
        grid=(GRID,),
        in_specs=layer_in_specs,
        out_specs=_row_spec((NPAD, H)),
        out_shape=jax.ShapeDtypeStruct((NPAD, H), jnp.float32),
    )(h, aggp[0], aggp[1], W1_0, b1_0.reshape(1, H), W_ih, b_ih2, W_hh, b_hh2,
      fuse_weight[0].reshape(1, 1))

    aggp = _sc_aggregate(h, edge_index, ew16)
    out = pl.pallas_call(
        _final_body,
        grid=(GRID,),
        in_specs=layer_in_specs + [_full_spec((2, H)), _full_spec((1, 2))],
        out_specs=_row_spec((NPAD, 2)),
        out_shape=jax.ShapeDtypeStruct((NPAD, 2), jnp.float32),
    )(h, aggp[0], aggp[1], W1_1, b1_1.reshape(1, H), W_ih, b_ih2, W_hh, b_hh2,
      fuse_weight[1].reshape(1, 1), W_out, b_out2)
    return out[:N]


# speedup vs baseline: 1.0737x; 1.0737x over previous
import functools

import jax
import jax.numpy as jnp
import numpy as np
from jax import lax
from jax.experimental import pallas as pl
from jax.experimental.pallas import tpu as pltpu
from jax.experimental.pallas import tpu_sc as plsc

N = 10000
E = 320000
H = 128
NC = 2
NS = 16
NW = NC * NS
EPW = E // NW
CH = 128
NPAD = 10112
RPT = NPAD // NS
ZR = 16
WB = 5

_sc_mesh = plsc.VectorSubcoreMesh(core_axis_name="c", subcore_axis_name="s")


@functools.partial(
    pl.kernel,
    out_type=jax.ShapeDtypeStruct((NC, NPAD, H), jnp.float32),
    mesh=_sc_mesh,
    scratch_types=[
        pltpu.VMEM((3, CH), jnp.int32),
        pltpu.VMEM((2, CH), jnp.int32),
        pltpu.VMEM((8, 2048), jnp.float32),
        pltpu.VMEM((2, CH, H), jnp.float32),
        pltpu.VMEM_SHARED((NPAD, H), jnp.float32),
        pltpu.SemaphoreType.DMA,
        pltpu.SemaphoreType.DMA,
        pltpu.SemaphoreType.DMA,
        pltpu.SemaphoreType.DMA,
        pltpu.SemaphoreType.DMA,
    ],
)
def _sc_aggregate(x_hbm, ei_hbm, ew_hbm, out_hbm,
                  sidx_v, didx_v, w_v, rows_v, acc_sh, gsem, ssem, dsem, wsem,
                  csem):
    c = lax.axis_index("c")
    s = lax.axis_index("s")
    wid = s * NC + c
    g = jnp.where(wid < 24, 10 * wid, 240 + 9 * (wid - 24))
    nk = jnp.where(wid < 24, 80, 72)
    ebase = g * 1024
    rowbase = g * 8

    def sidx_load(k):
        return pltpu.async_copy(
            ei_hbm.at[0, pl.ds(ebase + k * CH, CH)],
            sidx_v.at[lax.rem(k, 3)], ssem)

    def didx_load(k):
        return pltpu.async_copy(
            ei_hbm.at[1, pl.ds(ebase + k * CH, CH)],
            didx_v.at[lax.rem(k, 2)], dsem)

    def gather(k):
        return pltpu.async_copy(
            x_hbm.at[sidx_v.at[lax.rem(k, 3)]],
            rows_v.at[lax.rem(k, 2)], gsem)

    def mul_rows(b, kk):
        def mul(e, _):
            wsp = w_v[kk, pl.ds(e * 16, 16)]
            for j in range(8):
                rows_v[b, e, pl.ds(j * 16, 16)] = (
                    rows_v[b, e, pl.ds(j * 16, 16)] * wsp)
            return 0
        lax.fori_loop(0, CH, mul, 0, unroll=4)

    sidx_load(0).wait()
    gather(0)
    sidx_load(1)
    didx_load(0)

    def zb(k, _):
        rows_v[1, k // 8, pl.ds((k % 8) * 16, 16)] = jnp.zeros(
            (16,), jnp.float32)
        return 0
    lax.fori_loop(0, CH * 8, zb, 0)
    for t in range(4):
        pltpu.sync_copy(
            rows_v.at[1], acc_sh.at[pl.ds(s * RPT + t * 128, 128)])
    pltpu.sync_copy(
        rows_v.at[1, pl.ds(0, 120)], acc_sh.at[pl.ds(s * RPT + 512, 120)])
    plsc.subcore_barrier()

    def chunk(k, _):
        b = lax.rem(k, 2)
        @pl.when(lax.rem(k, 8) == 0)
        def _():
            pltpu.async_copy(
                ew_hbm.at[pl.ds(rowbase + (k // 8) * 8, 8)], w_v,
                wsem).wait()

        pltpu.make_async_copy(
            x_hbm.at[sidx_v.at[lax.rem(k, 3)]], rows_v.at[b], gsem).wait()

        @pl.when(k >= 1)
        def _():
            pltpu.make_async_copy(
                rows_v.at[1 - b], acc_sh.at[didx_v.at[1 - b]], csem).wait()
        @pl.when(k + 1 < nk)
        def _():
            pltpu.make_async_copy(
                ei_hbm.at[0, pl.ds(ebase + (k + 1) * CH, CH)],
                sidx_v.at[lax.rem(k + 1, 3)], ssem).wait()
            gather(k + 1)
            @pl.when(k + 2 < nk)
            def _():
                sidx_load(k + 2)

        mul_rows(b, lax.rem(k, 8))

        pltpu.make_async_copy(
            ei_hbm.at[1, pl.ds(ebase + k * CH, CH)],
            didx_v.at[b], dsem).wait()
        @pl.when(k + 1 < nk)
        def _():
            didx_load(k + 1)
        pltpu.async_copy(rows_v.at[b], acc_sh.at[didx_v.at[b]], csem,
                         add=True)
        return 0
    lax.fori_loop(0, nk, chunk, 0)
    pltpu.make_async_copy(
        rows_v.at[lax.rem(nk - 1, 2)],
        acc_sh.at[didx_v.at[lax.rem(nk - 1, 2)]], csem).wait()

    @pl.when(wid < 4)
    def _():
        eoff = 319488 + wid * CH
        pltpu.async_copy(ew_hbm.at[pl.ds(2496, 8)], w_v, wsem).wait()
        pltpu.async_copy(
            ei_hbm.at[0, pl.ds(eoff, CH)], sidx_v.at[0], ssem).wait()
        pltpu.async_copy(
            ei_hbm.at[1, pl.ds(eoff, CH)], didx_v.at[0], dsem).wait()
        pltpu.async_copy(
            x_hbm.at[sidx_v.at[0]], rows_v.at[0], gsem).wait()
        mul_rows(0, wid)
        pltpu.sync_copy(rows_v.at[0], acc_sh.at[didx_v.at[0]], add=True)
    plsc.subcore_barrier()

    for t in range(4):
        r0 = s * RPT + t * 128
        pltpu.sync_copy(acc_sh.at[pl.ds(r0, 128)], rows_v.at[0])
        pltpu.sync_copy(rows_v.at[0], out_hbm.at[c, pl.ds(r0, 128)])
    r0 = s * RPT + 512
    pltpu.sync_copy(acc_sh.at[pl.ds(r0, 120)], rows_v.at[0, pl.ds(0, 120)])
    pltpu.sync_copy(
        rows_v.at[0, pl.ds(0, 120)], out_hbm.at[c, pl.ds(r0, 120)])


def _dotT(a, b):
    return lax.dot_general(a, b, (((1,), (1,)), ((), ())),
                           preferred_element_type=jnp.float32)


BLK = 1264
GRID = NPAD // BLK


def _first_body(x_ref, w_ref, b_ref, o_ref):
    o_ref[...] = jnp.maximum(_dotT(x_ref[...], w_ref[...]) + b_ref[...], 0.0)


EROWS = 2560


def _rep16_body(w_ref, r_ref, o_ref):
    o_ref[...] = lax.dot_general(
        w_ref[...], r_ref[...], (((1,), (0,)), ((), ())),
        preferred_element_type=jnp.float32,
        precision=lax.Precision.HIGHEST)


def _gru_update(h, a0, a1, wl, bl, wih, bih, whh, bhh, fw):
    agg = a0 + a1
    xl = _dotT(agg, wl) + bl
    gi = _dotT(xl, wih) + bih
    gh = _dotT(h, whh) + bhh
    r = jax.nn.sigmoid(gi[:, :H] + gh[:, :H])
    z = jax.nn.sigmoid(gi[:, H:2 * H] + gh[:, H:2 * H])
    n = jnp.tanh(gi[:, 2 * H:] + r * gh[:, 2 * H:])
    return (1.0 - z) * n + z * h + fw * h


def _layer_body(h_ref, a0_ref, a1_ref, wl_ref, bl_ref, wih_ref, bih_ref,
                whh_ref, bhh_ref, fw_ref, o_ref):
    o_ref[...] = _gru_update(h_ref[...], a0_ref[...], a1_ref[...],
                             wl_ref[...], bl_ref[...], wih_ref[...],
                             bih_ref[...], whh_ref[...], bhh_ref[...],
                             fw_ref[0, 0])


def _final_body(h_ref, a0_ref, a1_ref, wl_ref, bl_ref, wih_ref, bih_ref,
                whh_ref, bhh_ref, fw_ref, wout_ref, bout_ref, o_ref):
    hn = _gru_update(h_ref[...], a0_ref[...], a1_ref[...], wl_ref[...],
                     bl_ref[...], wih_ref[...], bih_ref[...], whh_ref[...],
                     bhh_ref[...], fw_ref[0, 0])
    logits = _dotT(hn, wout_ref[...]) + bout_ref[...]
    m = jnp.max(logits, axis=-1, keepdims=True)
    lse = m + jnp.log(jnp.sum(jnp.exp(logits - m), axis=-1, keepdims=True))
    o_ref[...] = logits - lse


def _row_spec(shape):
    nd = len(shape)
    return pl.BlockSpec((BLK,) + tuple(shape[1:]),
                        lambda i, nd=nd: (i,) + (0,) * (nd - 1))


def _full_spec(shape):
    nd = len(shape)
    return pl.BlockSpec(tuple(shape), lambda i, nd=nd: (0,) * nd)


def kernel(x, edge_index, edge_weight, W_first, b_first, W1_0, b1_0, W1_1, b1_1,
           W_ih, W_hh, b_ih, b_hh, fuse_weight, W_out, b_out):
    ew_pad = jnp.concatenate(
        [edge_weight, jnp.zeros((EROWS * 128 - E,), jnp.float32)])
    rep = (jnp.arange(2048, dtype=jnp.int32)[None, :] // 16
           == jnp.arange(128, dtype=jnp.int32)[:, None]).astype(jnp.float32)
    ew16 = pl.pallas_call(
        _rep16_body,
        grid=(EROWS // 160,),
        in_specs=[pl.BlockSpec((160, 128), lambda i: (i, 0)),
                  pl.BlockSpec((128, 2048), lambda i: (0, 0))],
        out_specs=pl.BlockSpec((160, 128 * 16), lambda i: (i, 0)),
        out_shape=jax.ShapeDtypeStruct((EROWS, 128 * 16), jnp.float32),
    )(ew_pad.reshape(EROWS, 128), rep)
    b_first2 = b_first.reshape(1, H)
    b_ih2 = b_ih.reshape(1, 3 * H)
    b_hh2 = b_hh.reshape(1, 3 * H)
    b_out2 = b_out.reshape(1, 2)

    x0 = pl.pallas_call(
        _first_body,
        grid=(GRID,),
        in_specs=[_row_spec((N, H)), _full_spec((H, H)), _full_spec((1, H))],
        out_specs=_row_spec((NPAD, H)),
        out_shape=jax.ShapeDtypeStruct((NPAD, H), jnp.float32),
    )(x, W_first, b_first2)

    layer_in_specs = [
        _row_spec((NPAD, H)),
        _row_spec((NPAD, H)),
        _row_spec((NPAD, H)),
        _full_spec((H, H)),
        _full_spec((1, H)),
        _full_spec((3 * H, H)),
        _full_spec((1, 3 * H)),
        _full_spec((3 * H, H)),
        _full_spec((1, 3 * H)),
        _full_spec((1, 1)),
    ]

    h = x0
    aggp = _sc_aggregate(h, edge_index, ew16)
    h = pl.pallas_call(
        _layer_body,
        grid=(GRID,),
        in_specs=layer_in_specs,
        out_specs=_row_spec((NPAD, H)),
        out_shape=jax.ShapeDtypeStruct((NPAD, H), jnp.float32),
    )(h, aggp[0], aggp[1], W1_0, b1_0.reshape(1, H), W_ih, b_ih2, W_hh, b_hh2,
      fuse_weight[0].reshape(1, 1))

    aggp = _sc_aggregate(h, edge_index, ew16)
    out = pl.pallas_call(
        _final_body,
        grid=(GRID,),
        in_specs=layer_in_specs + [_full_spec((2, H)), _full_spec((1, 2))],
        out_specs=_row_spec((NPAD, 2)),
        out_shape=jax.ShapeDtypeStruct((NPAD, 2), jnp.float32),
    )(h, aggp[0], aggp[1], W1_1, b1_1.reshape(1, H), W_ih, b_ih2, W_hh, b_hh2,
      fuse_weight[1].reshape(1, 1), W_out, b_out2)
    return out[:N]
